# baseline (speedup 1.0000x reference)
import jax
import jax.numpy as jnp
from jax.experimental import pallas as pl
from jax.experimental.pallas import tpu as pltpu

_ANCHOR_W = (116.0, 156.0, 373.0)
_ANCHOR_H = (90.0, 198.0, 326.0)
_A = 3
_CLS = 80
_HB = 16


def _decode_kernel(x_ref, o_ref):
    n, _, hb, w = x_ref.shape
    h0 = (pl.program_id(0) * hb).astype(jnp.float32)
    a = pl.program_id(1)
    aw = jnp.where(
        a == 0, jnp.float32(_ANCHOR_W[0]),
        jnp.where(a == 1, jnp.float32(_ANCHOR_W[1]), jnp.float32(_ANCHOR_W[2])),
    )
    ah = jnp.where(
        a == 0, jnp.float32(_ANCHOR_H[0]),
        jnp.where(a == 1, jnp.float32(_ANCHOR_H[1]), jnp.float32(_ANCHOR_H[2])),
    )
    gx = jax.lax.broadcasted_iota(jnp.int32, (hb, w), 1).astype(jnp.float32)
    gy = jax.lax.broadcasted_iota(jnp.int32, (hb, w), 0).astype(jnp.float32) + h0
    bx = jax.nn.sigmoid(x_ref[:, 0]) + gx[None]
    by = jax.nn.sigmoid(x_ref[:, 1]) + gy[None]
    bw = x_ref[:, 2] * aw
    bh = x_ref[:, 3] * ah
    obj = x_ref[:, 4]
    logits = x_ref[:, 5 : 5 + _CLS]
    score = logits * obj[:, None]
    s = score.reshape(n * _CLS, hb, w)
    smax = jnp.max(s, axis=0)
    idx = jax.lax.broadcasted_iota(jnp.int32, (n * _CLS, hb, w), 0).astype(
        jnp.float32
    )
    sarg = jnp.min(
        jnp.where(s == smax[None], idx, jnp.float32(n * _CLS)), axis=0
    )
    o_ref[:, 0] = bx
    o_ref[:, 1] = by
    o_ref[:, 2] = bw
    o_ref[:, 3] = bh
    o_ref[:, 4] = jnp.broadcast_to(smax[None], (n, hb, w))
    o_ref[:, 5] = jnp.broadcast_to(sarg[None], (n, hb, w))


def kernel(x):
    n, c, h, w = x.shape
    ca = 5 + _CLS
    return pl.pallas_call(
        _decode_kernel,
        grid=(h // _HB, _A),
        in_specs=[pl.BlockSpec((n, ca, _HB, w), lambda i, a: (0, a, i, 0))],
        out_specs=pl.BlockSpec((n, 6, _HB, w), lambda i, a: (0, a, i, 0)),
        out_shape=jax.ShapeDtypeStruct((n, _A * 6, h, w), x.dtype),
        compiler_params=pltpu.CompilerParams(
            dimension_semantics=("parallel", "parallel"),
            vmem_limit_bytes=64 * 1024 * 1024,
        ),
        name="anchor_decode",
    )(x)


# speedup vs baseline: 1.1815x; 1.1815x over previous
import jax
import jax.numpy as jnp
from jax.experimental import pallas as pl
from jax.experimental.pallas import tpu as pltpu

_ANCHOR_W = (116.0, 156.0, 373.0)
_ANCHOR_H = (90.0, 198.0, 326.0)
_A = 3
_CLS = 80
_CHUNKS = ((0, 8), (8, 8), (16, 16), (32, 16), (48, 16), (64, 16),
           (80, 16), (96, 16), (112, 8), (120, 8))


def _compute(buf, ob, off, sz):
    n = buf.shape[0]
    w = buf.shape[3]
    gx = jax.lax.broadcasted_iota(jnp.int32, (sz, w), 1).astype(jnp.float32)
    gy = jax.lax.broadcasted_iota(jnp.int32, (sz, w), 0).astype(jnp.float32) + float(off)
    for a in range(_A):
        base = a * (5 + _CLS)
        bx = jax.nn.sigmoid(buf[:, base + 0, 0:sz]) + gx[None]
        by = jax.nn.sigmoid(buf[:, base + 1, 0:sz]) + gy[None]
        bw = buf[:, base + 2, 0:sz] * _ANCHOR_W[a]
        bh = buf[:, base + 3, 0:sz] * _ANCHOR_H[a]
        obj = buf[:, base + 4, 0:sz]
        logits = buf[:, base + 5 : base + 5 + _CLS, 0:sz]
        score = logits * obj[:, None]
        s = score.reshape(n * _CLS, sz, w)
        smax = jnp.max(s, axis=0)
        idx = jax.lax.broadcasted_iota(jnp.int32, (n * _CLS, sz, w), 0).astype(
            jnp.float32
        )
        sarg = jnp.min(
            jnp.where(s == smax[None], idx, jnp.float32(n * _CLS)), axis=0
        )
        ob[:, a * 6 + 0, 0:sz] = bx
        ob[:, a * 6 + 1, 0:sz] = by
        ob[:, a * 6 + 2, 0:sz] = bw
        ob[:, a * 6 + 3, 0:sz] = bh
        ob[:, a * 6 + 4, 0:sz] = jnp.broadcast_to(smax[None], (n, sz, w))
        ob[:, a * 6 + 5, 0:sz] = jnp.broadcast_to(sarg[None], (n, sz, w))


def _decode_kernel(x_hbm, o_hbm, b0, b1, ob0, ob1, insem, outsem):
    bufs = (b0, b1)
    obufs = (ob0, ob1)

    def in_copy(k):
        off, sz = _CHUNKS[k]
        return pltpu.make_async_copy(
            x_hbm.at[:, :, pl.ds(off, sz), :],
            bufs[k % 2].at[:, :, pl.ds(0, sz), :],
            insem.at[k % 2],
        )

    def out_copy(k):
        off, sz = _CHUNKS[k]
        return pltpu.make_async_copy(
            obufs[k % 2].at[:, :, pl.ds(0, sz), :],
            o_hbm.at[:, :, pl.ds(off, sz), :],
            outsem.at[k % 2],
        )

    in_copy(0).start()
    in_copy(1).start()
    nk = len(_CHUNKS)
    for k in range(nk):
        off, sz = _CHUNKS[k]
        slot = k % 2
        in_copy(k).wait()
        if k >= 2:
            out_copy(k - 2).wait()
        _compute(bufs[slot], obufs[slot], off, sz)
        out_copy(k).start()
        if k + 2 < nk:
            in_copy(k + 2).start()
    out_copy(nk - 2).wait()
    out_copy(nk - 1).wait()


def kernel(x):
    n, c, h, w = x.shape
    return pl.pallas_call(
        _decode_kernel,
        in_specs=[pl.BlockSpec(memory_space=pl.ANY)],
        out_specs=pl.BlockSpec(memory_space=pl.ANY),
        out_shape=jax.ShapeDtypeStruct((n, _A * 6, h, w), x.dtype),
        scratch_shapes=[
            pltpu.VMEM((n, c, 16, w), jnp.float32),
            pltpu.VMEM((n, c, 16, w), jnp.float32),
            pltpu.VMEM((n, _A * 6, 16, w), jnp.float32),
            pltpu.VMEM((n, _A * 6, 16, w), jnp.float32),
            pltpu.SemaphoreType.DMA((2,)),
            pltpu.SemaphoreType.DMA((2,)),
        ],
        compiler_params=pltpu.CompilerParams(
            vmem_limit_bytes=64 * 1024 * 1024,
        ),
        name="anchor_decode_manual",
    )(x)
